# baseline (speedup 1.0000x reference)
import functools

import jax
import jax.numpy as jnp
from jax import lax
from jax.experimental import pallas as pl
from jax.experimental.pallas import tpu as pltpu
from jax.experimental.pallas import tpu_sc as plsc

VOCAB = 1000000
EMB_DIM = 64
OUT_DIM = 1000
BATCH = 4096

NC = 2
NS = 16
NW = NC * NS
ROWS_PER_W = BATCH // NW
L = 200
HALF = 100
NBUF = 3
VREGS = EMB_DIM // 16


def _sc_gather_sum(idx2d, table):
    mesh = plsc.VectorSubcoreMesh(core_axis_name="c", subcore_axis_name="s")

    @functools.partial(
        pl.kernel,
        out_type=jax.ShapeDtypeStruct((BATCH, EMB_DIM), jnp.float32),
        mesh=mesh,
        scratch_types=[
            pltpu.VMEM((2 * ROWS_PER_W, HALF), jnp.int32),
            pltpu.VMEM((NBUF, L, EMB_DIM), jnp.float32),
            pltpu.VMEM((ROWS_PER_W, EMB_DIM), jnp.float32),
        ] + [pltpu.SemaphoreType.DMA] * NBUF,
        compiler_params=pltpu.CompilerParams(use_tc_tiling_on_sc=False),
    )
    def k(idx_hbm, table_hbm, out_hbm, idx_v, buf_v, acc_v, *sems):
        wid = lax.axis_index("s") * NC + lax.axis_index("c")
        pltpu.sync_copy(idx_hbm.at[pl.ds(wid * 2 * ROWS_PER_W, 2 * ROWS_PER_W)],
                        idx_v)

        def gather_copies(t, b):
            return [
                pltpu.make_async_copy(
                    table_hbm.at[idx_v.at[2 * t + h]],
                    buf_v.at[b, pl.ds(h * HALF, HALF)],
                    sems[b],
                )
                for h in range(2)
            ]

        for b in range(NBUF):
            for cp in gather_copies(b, b):
                cp.start()

        def outer(g, _):
            for b in range(NBUF):
                t = g * NBUF + b
                for cp in gather_copies(t, b):
                    cp.wait()

                def body(i, vs):
                    return tuple(
                        vs[c] + buf_v[b, i, pl.ds(16 * c, 16)]
                        for c in range(VREGS)
                    )
                vs = lax.fori_loop(
                    0, L, body,
                    tuple(jnp.zeros((16,), jnp.float32) for _ in range(VREGS)))
                for c in range(VREGS):
                    acc_v[t, pl.ds(16 * c, 16)] = vs[c]

                @pl.when(t + NBUF < ROWS_PER_W)
                def _():
                    for cp in gather_copies(t + NBUF, b):
                        cp.start()
            return ()

        lax.fori_loop(0, ROWS_PER_W // NBUF, outer, ())
        for t in range((ROWS_PER_W // NBUF) * NBUF, ROWS_PER_W):
            b = t % NBUF
            for cp in gather_copies(t, b):
                cp.wait()

            def body(i, vs):
                return tuple(
                    vs[c] + buf_v[b, i, pl.ds(16 * c, 16)]
                    for c in range(VREGS)
                )
            vs = lax.fori_loop(
                0, L, body,
                tuple(jnp.zeros((16,), jnp.float32) for _ in range(VREGS)))
            for c in range(VREGS):
                acc_v[t, pl.ds(16 * c, 16)] = vs[c]

        pltpu.sync_copy(acc_v, out_hbm.at[pl.ds(wid * ROWS_PER_W, ROWS_PER_W)])

    return k(idx2d, table)


def _tc_norm_linear_body(x_ref, w_ref, b_ref, o_ref):
    x = x_ref[...]
    ss = jnp.sum(x * x, axis=1, keepdims=True)
    inv = lax.rsqrt(jnp.maximum(ss, 1e-24))
    xn = x * inv
    o_ref[...] = lax.dot_general(
        xn, w_ref[...], (((1,), (1,)), ((), ())),
        preferred_element_type=jnp.float32) + b_ref[...]


def _tc_norm_linear(sums, W, b):
    BM = 512
    return pl.pallas_call(
        _tc_norm_linear_body,
        grid=(BATCH // BM,),
        in_specs=[
            pl.BlockSpec((BM, EMB_DIM), lambda i: (i, 0)),
            pl.BlockSpec((OUT_DIM, EMB_DIM), lambda i: (0, 0)),
            pl.BlockSpec((1, OUT_DIM), lambda i: (0, 0)),
        ],
        out_specs=pl.BlockSpec((BM, OUT_DIM), lambda i: (i, 0)),
        out_shape=jax.ShapeDtypeStruct((BATCH, OUT_DIM), jnp.float32),
    )(sums, W, b.reshape(1, OUT_DIM))


@jax.jit
def kernel(name_idxs, name_len, desc_idxs, desc_len, union_idxs, union_len,
           table, W, b):
    idx2d = union_idxs.astype(jnp.int32).reshape(2 * BATCH, HALF)
    sums = _sc_gather_sum(idx2d, table)
    return _tc_norm_linear(sums, W, b)
